# baseline (speedup 1.0000x reference)
import functools

import jax
import jax.numpy as jnp
from jax import lax
from jax.experimental import pallas as pl
from jax.experimental.pallas import tpu as pltpu
from jax.experimental.pallas import tpu_sc as plsc

D_MODEL = 64
NC, NS = 2, 16
NW = NC * NS
NBUF = 3
LANES = 16


def kernel(inputs, token_table, pos_table):
    nseq, slen = inputs.shape
    seqs_per_w = nseq // NW
    ids_per_w = seqs_per_w * slen
    groups = seqs_per_w // NBUF
    nvec = D_MODEL // LANES
    splits = [(0, 128), (128, slen - 128)]
    mesh = plsc.VectorSubcoreMesh(core_axis_name="c", subcore_axis_name="s")

    @functools.partial(
        pl.kernel,
        out_type=jax.ShapeDtypeStruct((nseq, slen, D_MODEL), jnp.float32),
        mesh=mesh,
        scratch_types=(
            [pltpu.VMEM((ids_per_w,), jnp.int32),
             pltpu.VMEM((slen * D_MODEL,), jnp.float32)]
            + [pltpu.VMEM((slen, D_MODEL), jnp.float32)] * NBUF
            + [pltpu.SemaphoreType.DMA] * (2 * NBUF)
        ),
        compiler_params=pltpu.CompilerParams(use_tc_tiling_on_sc=False),
    )
    def k(ids_hbm, tok_tab_hbm, pos_tab_hbm, out_hbm, ti_all, pos_v, *rest):
        bufs = rest[:NBUF]
        sem_g = rest[NBUF:2 * NBUF]
        sem_s = rest[2 * NBUF:]
        wid = lax.axis_index("s") * NC + lax.axis_index("c")
        seq0 = wid * seqs_per_w

        pltpu.sync_copy(ids_hbm.at[pl.ds(seq0 * slen, ids_per_w)], ti_all)
        pltpu.sync_copy(pos_tab_hbm.at[pl.ds(0, slen * D_MODEL)], pos_v)

        def group(g, carry):
            descs = []
            for b in range(NBUF):
                c = g * NBUF + b

                @pl.when(g > 0)
                def _wait_prev_store(b=b):
                    pltpu.make_async_copy(
                        bufs[b], out_hbm.at[0], sem_s[b]).wait()

                descs.append(tuple(
                    pltpu.async_copy(
                        tok_tab_hbm.at[ti_all.at[pl.ds(c * slen + o, w)]],
                        bufs[b].at[pl.ds(o, w)], sem_g[b])
                    for (o, w) in splits))
            for b in range(NBUF):
                c = g * NBUF + b
                for d in descs[b]:
                    d.wait()
                buf = bufs[b]

                def addrow(j, carry2, buf=buf):
                    for v in range(nvec):
                        sl = pl.ds(v * LANES, LANES)
                        buf[j, sl] = (buf[j, sl]
                                      + pos_v[pl.ds(j * D_MODEL + v * LANES,
                                                    LANES)])
                    return carry2

                lax.fori_loop(0, slen, addrow, 0)
                pltpu.async_copy(buf, out_hbm.at[seq0 + c], sem_s[b])
            return carry

        lax.fori_loop(0, groups, group, 0)
        for b in range(NBUF):
            pltpu.make_async_copy(bufs[b], out_hbm.at[0], sem_s[b]).wait()

    flat_ids = inputs.reshape(nseq * slen)
    flat_pos = pos_table.reshape(pos_table.shape[0] * D_MODEL)
    return k(flat_ids, token_table, flat_pos)


# speedup vs baseline: 1.0034x; 1.0034x over previous
import functools

import jax
import jax.numpy as jnp
from jax import lax
from jax.experimental import pallas as pl
from jax.experimental.pallas import tpu as pltpu
from jax.experimental.pallas import tpu_sc as plsc

D_MODEL = 64
NC, NS = 2, 16
NW = NC * NS
NBUF = 3
LANES = 16


def kernel(inputs, token_table, pos_table):
    nseq, slen = inputs.shape
    seqs_per_w = nseq // NW
    ids_per_w = seqs_per_w * slen
    groups = seqs_per_w // NBUF
    nvec = D_MODEL // LANES
    splits = [(0, 128), (128, slen - 128)]
    mesh = plsc.VectorSubcoreMesh(core_axis_name="c", subcore_axis_name="s")

    @functools.partial(
        pl.kernel,
        out_type=jax.ShapeDtypeStruct((nseq, slen, D_MODEL), jnp.float32),
        mesh=mesh,
        scratch_types=(
            [pltpu.VMEM((ids_per_w,), jnp.int32),
             pltpu.VMEM((slen * D_MODEL,), jnp.float32)]
            + [pltpu.VMEM((slen, D_MODEL), jnp.float32)] * NBUF
            + [pltpu.SemaphoreType.DMA] * (2 * NBUF)
        ),
        compiler_params=pltpu.CompilerParams(use_tc_tiling_on_sc=False),
    )
    def k(ids_hbm, tok_tab_hbm, pos_tab_hbm, out_hbm, ti_all, pos_v, *rest):
        bufs = rest[:NBUF]
        sem_g = rest[NBUF:2 * NBUF]
        sem_s = rest[2 * NBUF:]
        wid = lax.axis_index("s") * NC + lax.axis_index("c")
        seq0 = wid * seqs_per_w

        pltpu.sync_copy(ids_hbm.at[pl.ds(seq0 * slen, ids_per_w)], ti_all)
        pltpu.sync_copy(pos_tab_hbm.at[pl.ds(0, slen * D_MODEL)], pos_v)

        def group(g, carry):
            descs = []
            for b in range(NBUF):
                c = g * NBUF + b

                @pl.when(g > 0)
                def _wait_prev_store(b=b):
                    pltpu.make_async_copy(
                        bufs[b], out_hbm.at[0], sem_s[b]).wait()

                descs.append(tuple(
                    pltpu.async_copy(
                        tok_tab_hbm.at[ti_all.at[pl.ds(c * slen + o, w)]],
                        bufs[b].at[pl.ds(o, w)], sem_g[b])
                    for (o, w) in splits))
            for b in range(NBUF):
                c = g * NBUF + b
                for d in descs[b]:
                    d.wait()
                buf = bufs[b]

                def addrow(j, carry2, buf=buf):
                    for v in range(nvec):
                        sl = pl.ds(v * LANES, LANES)
                        buf[j, sl] = (buf[j, sl]
                                      + pos_v[pl.ds(j * D_MODEL + v * LANES,
                                                    LANES)])
                    return carry2

                lax.fori_loop(0, slen, addrow, 0)
                pltpu.async_copy(buf, out_hbm.at[seq0 + c], sem_s[b])
            return carry

        lax.fori_loop(0, groups, group, 0)
        for b in range(NBUF):
            pltpu.make_async_copy(bufs[b], out_hbm.at[0], sem_s[b]).wait()

    ids_x = lax.optimization_barrier(jnp.bitwise_xor(inputs, 0x5A5A5A5A))
    flat_ids = jnp.bitwise_xor(ids_x, 0x5A5A5A5A).reshape(nseq * slen)
    pos_i = pos_table.view(jnp.int32)
    pos_x = lax.optimization_barrier(jnp.bitwise_xor(pos_i, 0x5A5A5A5A))
    flat_pos = jnp.bitwise_xor(pos_x, 0x5A5A5A5A).reshape(
        pos_table.shape[0] * D_MODEL).view(jnp.float32)
    return k(flat_ids, token_table, flat_pos)
